# Initial kernel scaffold; baseline (speedup 1.0000x reference)
import jax
import jax.numpy as jnp
from jax.experimental import pallas as pl


def kernel(x, edge_index, W1, b1, W2, b2, W3, b3):
    raise NotImplementedError("write your pallas kernel here")


# speedup vs baseline: 18.0504x; 18.0504x over previous
import functools

import jax
import jax.numpy as jnp
from jax import lax
from jax.experimental import pallas as pl
from jax.experimental.pallas import tpu as pltpu
from jax.experimental.pallas import tpu_sc as plsc

N = 10000
E = 320000
NC = 2
NS = 16
NW = NC * NS
EPW = E // NW
B = 80
STEPS = EPW // B
RPT = 632
RPT_LAST = N - RPT * (NS - 1)

_MESH = plsc.VectorSubcoreMesh(core_axis_name="c", subcore_axis_name="s")


@functools.partial(
    pl.kernel,
    out_type=jax.ShapeDtypeStruct((NC, N), jnp.float32),
    mesh=_MESH,
    scratch_types=[
        pltpu.VMEM((B,), jnp.float32),
        pltpu.VMEM((STEPS, B), jnp.int32),
        pltpu.VMEM((640,), jnp.float32),
        pltpu.VMEM_SHARED((N,), jnp.float32),
    ],
)
def _deg(dst_hbm, deg_hbm, ones_v, idx_v, zbuf, acc):
    c = lax.axis_index("c")
    s = lax.axis_index("s")
    wid = c * NS + s
    one = jnp.ones((16,), jnp.float32)
    zero = jnp.zeros((16,), jnp.float32)
    for k in range(B // 16):
        ones_v[pl.ds(16 * k, 16)] = one

    @pl.when(s == 0)
    def _zero_acc():
        for k in range(40):
            zbuf[pl.ds(16 * k, 16)] = zero
        for k in range(15):
            pltpu.sync_copy(zbuf, acc.at[pl.ds(640 * k, 640)])
        pltpu.sync_copy(zbuf.at[pl.ds(0, 400)], acc.at[pl.ds(9600, 400)])

    pltpu.sync_copy(dst_hbm.at[wid], idx_v)
    plsc.subcore_barrier()

    def step(i, carry):
        pltpu.sync_copy(ones_v, acc.at[idx_v.at[i]], add=True)
        return carry

    lax.fori_loop(0, STEPS, step, 0)
    plsc.subcore_barrier()

    @pl.when(s == 0)
    def _copy_out():
        pltpu.sync_copy(acc, deg_hbm.at[c])


def _make_agg(D):
    @functools.partial(
        pl.kernel,
        out_type=jax.ShapeDtypeStruct((NC, N, D), jnp.float32),
        mesh=_MESH,
        scratch_types=[
            pltpu.VMEM((STEPS, B), jnp.int32),
            pltpu.VMEM((STEPS, B), jnp.int32),
            pltpu.VMEM((B, D), jnp.float32),
            pltpu.VMEM_SHARED((N, D), jnp.float32),
            pltpu.SemaphoreType.DMA,
        ],
        compiler_params=pltpu.CompilerParams(use_tc_tiling_on_sc=False),
    )
    def agg(h_hbm, src_hbm, dst_hbm, out_hbm, sidx, didx, rows, acc, gsem):
        c = lax.axis_index("c")
        s = lax.axis_index("s")
        wid = c * NS + s
        zero = jnp.zeros((16,), jnp.float32)

        def zrow(r, carry):
            def zcol(k, carry2):
                rows[r, pl.ds(k * 16, 16)] = zero
                return carry2
            return lax.fori_loop(0, D // 16, zcol, carry)

        lax.fori_loop(0, B, zrow, 0)
        base = s * RPT

        def fill_zero(nrows):
            for k in range(nrows // B):
                pltpu.sync_copy(rows, acc.at[pl.ds(base + k * B, B)])
            rem = nrows % B
            pltpu.sync_copy(rows.at[pl.ds(0, rem)],
                            acc.at[pl.ds(base + nrows - rem, rem)])

        @pl.when(s < NS - 1)
        def _zfull():
            fill_zero(RPT)

        @pl.when(s == NS - 1)
        def _zlast():
            fill_zero(RPT_LAST)

        pltpu.sync_copy(src_hbm.at[wid], sidx)
        pltpu.sync_copy(dst_hbm.at[wid], didx)
        plsc.subcore_barrier()

        def step(i, carry):
            pltpu.async_copy(h_hbm.at[sidx.at[i]], rows, gsem).wait()
            pltpu.sync_copy(rows, acc.at[didx.at[i]], add=True)
            return carry

        lax.fori_loop(0, STEPS, step, 0)
        plsc.subcore_barrier()

        @pl.when(s < NS - 1)
        def _cfull():
            pltpu.sync_copy(acc.at[pl.ds(base, RPT)],
                            out_hbm.at[c, pl.ds(base, RPT)])

        @pl.when(s == NS - 1)
        def _clast():
            pltpu.sync_copy(acc.at[pl.ds(base, RPT_LAST)],
                            out_hbm.at[c, pl.ds(base, RPT_LAST)])

    return agg


_agg128 = _make_agg(128)
_agg64 = _make_agg(64)


GB = 2000
GRID = N // GB
_DOT = dict(preferred_element_type=jnp.float32, precision=lax.Precision.HIGHEST)


def _mm_body(x_ref, w_ref, o_ref):
    o_ref[...] = lax.dot_general(x_ref[...], w_ref[...],
                                 (((1,), (0,)), ((), ())), **_DOT)


def _scale_body(deg_ref, m_ref, o_ref):
    dinv = lax.rsqrt(1.0 + deg_ref[0] + deg_ref[1])
    o_ref[...] = m_ref[...] * dinv


def _fused_body(agg_ref, h_ref, deg_ref, b_ref, w_ref, o_ref):
    dinv = lax.rsqrt(1.0 + deg_ref[0] + deg_ref[1])
    t = jnp.maximum(dinv * (agg_ref[0] + agg_ref[1] + h_ref[...]) + b_ref[...],
                    0.0)
    o_ref[...] = lax.dot_general(t * dinv, w_ref[...],
                                 (((1,), (0,)), ((), ())), **_DOT)


def _final_body(agg_ref, h_ref, deg_ref, b_ref, o_ref):
    dinv = lax.rsqrt(1.0 + deg_ref[0] + deg_ref[1])
    o_ref[...] = dinv * (agg_ref[0] + agg_ref[1] + h_ref[...]) + b_ref[...]


def _row_spec(d):
    return pl.BlockSpec((GB, d), lambda i: (i, 0))


_DEG_SPEC = pl.BlockSpec((NC, GB, 1), lambda i: (0, i, 0))


def _agg_spec(d):
    return pl.BlockSpec((NC, GB, d), lambda i: (0, i, 0))


def _w_spec(din, dout):
    return pl.BlockSpec((din, dout), lambda i: (0, 0))


def _b_spec(d):
    return pl.BlockSpec((1, d), lambda i: (0, 0))


_mm = pl.pallas_call(
    _mm_body, grid=(GRID,),
    in_specs=[_row_spec(128), _w_spec(128, 128)],
    out_specs=_row_spec(128),
    out_shape=jax.ShapeDtypeStruct((N, 128), jnp.float32),
)

_scale = pl.pallas_call(
    _scale_body, grid=(GRID,),
    in_specs=[_DEG_SPEC, _row_spec(128)],
    out_specs=_row_spec(128),
    out_shape=jax.ShapeDtypeStruct((N, 128), jnp.float32),
)


def _make_fused(dout):
    return pl.pallas_call(
        _fused_body, grid=(GRID,),
        in_specs=[_agg_spec(128), _row_spec(128), _DEG_SPEC, _b_spec(128),
                  _w_spec(128, dout)],
        out_specs=_row_spec(dout),
        out_shape=jax.ShapeDtypeStruct((N, dout), jnp.float32),
    )


_fused128 = _make_fused(128)
_fused64 = _make_fused(64)

_final = pl.pallas_call(
    _final_body, grid=(GRID,),
    in_specs=[_agg_spec(64), _row_spec(64), _DEG_SPEC, _b_spec(64)],
    out_specs=_row_spec(64),
    out_shape=jax.ShapeDtypeStruct((N, 64), jnp.float32),
)


def kernel(x, edge_index, W1, b1, W2, b2, W3, b3):
    src3 = edge_index[0].astype(jnp.int32).reshape(NW, STEPS, B)
    dst3 = edge_index[1].astype(jnp.int32).reshape(NW, STEPS, B)
    deg2 = _deg(dst3)
    deg3 = deg2.reshape(NC, N, 1)
    mat1 = _mm(x, W1)
    h1p = _scale(deg3, mat1)
    agg1 = _agg128(h1p, src3, dst3)
    h2p = _fused128(agg1, h1p, deg3, b1.reshape(1, -1), W2)
    agg2 = _agg128(h2p, src3, dst3)
    h3p = _fused64(agg2, h2p, deg3, b2.reshape(1, -1), W3)
    agg3 = _agg64(h3p, src3, dst3)
    return _final(agg3, h3p, deg3, b3.reshape(1, -1))


# speedup vs baseline: 23.7247x; 1.3144x over previous
import functools

import jax
import jax.numpy as jnp
from jax import lax
from jax.experimental import pallas as pl
from jax.experimental.pallas import tpu as pltpu
from jax.experimental.pallas import tpu_sc as plsc

N = 10000
E = 320000
NC = 2
NS = 16
NW = NC * NS
B = 96
STEPS = 106
PAIRS = STEPS // 2
EP = NW * STEPS * B
NA = 10112
RPT = NA // NS
_CHUNKS = [B] * (RPT // B) + ([RPT % B] if RPT % B else [])

_MESH = plsc.VectorSubcoreMesh(core_axis_name="c", subcore_axis_name="s")
_SC_PARAMS = pltpu.CompilerParams(use_tc_tiling_on_sc=False)


@functools.partial(
    pl.kernel,
    out_type=jax.ShapeDtypeStruct((NC, N), jnp.float32),
    mesh=_MESH,
    scratch_types=[
        pltpu.VMEM((B,), jnp.float32),
        pltpu.VMEM((STEPS, B), jnp.int32),
        pltpu.VMEM((640,), jnp.float32),
        pltpu.VMEM_SHARED((NA,), jnp.float32),
    ],
    compiler_params=_SC_PARAMS,
)
def _deg(dst_hbm, deg_hbm, ones_v, idx_v, zbuf, acc):
    c = lax.axis_index("c")
    s = lax.axis_index("s")
    wid = c * NS + s
    one = jnp.ones((16,), jnp.float32)
    zero = jnp.zeros((16,), jnp.float32)
    for k in range(B // 16):
        ones_v[pl.ds(16 * k, 16)] = one

    @pl.when(s == 0)
    def _zero_acc():
        for k in range(40):
            zbuf[pl.ds(16 * k, 16)] = zero
        for k in range(NA // 640):
            pltpu.sync_copy(zbuf, acc.at[pl.ds(640 * k, 640)])
        rem = NA % 640
        if rem:
            pltpu.sync_copy(zbuf.at[pl.ds(0, rem)],
                            acc.at[pl.ds(NA - rem, rem)])

    pltpu.sync_copy(dst_hbm.at[wid], idx_v)
    plsc.subcore_barrier()

    def step(i, carry):
        pltpu.sync_copy(ones_v, acc.at[idx_v.at[i]], add=True)
        return carry

    lax.fori_loop(0, STEPS, step, 0)
    plsc.subcore_barrier()

    @pl.when(s == 0)
    def _copy_out():
        pltpu.sync_copy(acc.at[pl.ds(0, N)], deg_hbm.at[c])


def _make_agg(D):
    @functools.partial(
        pl.kernel,
        out_type=jax.ShapeDtypeStruct((NC, NA, D), jnp.float32),
        mesh=_MESH,
        scratch_types=[
            pltpu.VMEM((STEPS, B), jnp.int32),
            pltpu.VMEM((STEPS, B), jnp.int32),
            pltpu.VMEM((B, D), jnp.float32),
            pltpu.VMEM((B, D), jnp.float32),
            pltpu.VMEM_SHARED((NA, D), jnp.float32),
            pltpu.SemaphoreType.DMA,
            pltpu.SemaphoreType.DMA,
        ],
        compiler_params=_SC_PARAMS,
    )
    def agg(h_hbm, src_hbm, dst_hbm, out_hbm, sidx, didx, rows0, rows1, acc,
            gsem0, gsem1):
        c = lax.axis_index("c")
        s = lax.axis_index("s")
        wid = c * NS + s
        zero = jnp.zeros((16,), jnp.float32)

        def zrow(r, carry):
            def zcol(k, carry2):
                rows0[r, pl.ds(k * 16, 16)] = zero
                return carry2
            return lax.fori_loop(0, D // 16, zcol, carry)

        lax.fori_loop(0, B, zrow, 0)
        base = s * RPT
        off = 0
        for n in _CHUNKS:
            pltpu.sync_copy(rows0.at[pl.ds(0, n)],
                            acc.at[pl.ds(base + off, n)])
            off += n
        pltpu.sync_copy(src_hbm.at[wid], sidx)
        pltpu.sync_copy(dst_hbm.at[wid], didx)
        plsc.subcore_barrier()

        pltpu.async_copy(h_hbm.at[sidx.at[0]], rows0, gsem0)

        def pair(p, carry):
            i0 = 2 * p
            pltpu.make_async_copy(h_hbm.at[sidx.at[i0]], rows0, gsem0).wait()
            pltpu.async_copy(h_hbm.at[sidx.at[i0 + 1]], rows1, gsem1)
            pltpu.sync_copy(rows0, acc.at[didx.at[i0]], add=True)
            pltpu.make_async_copy(h_hbm.at[sidx.at[i0 + 1]], rows1,
                                  gsem1).wait()

            @pl.when(p < PAIRS - 1)
            def _prefetch():
                pltpu.async_copy(h_hbm.at[sidx.at[i0 + 2]], rows0, gsem0)

            pltpu.sync_copy(rows1, acc.at[didx.at[i0 + 1]], add=True)
            return carry

        lax.fori_loop(0, PAIRS, pair, 0)
        plsc.subcore_barrier()
        off = 0
        for n in _CHUNKS:
            pltpu.sync_copy(acc.at[pl.ds(base + off, n)],
                            out_hbm.at[c, pl.ds(base + off, n)])
            off += n

    return agg


_agg128 = _make_agg(128)
_agg64 = _make_agg(64)


GB = 2000
GRID = N // GB
_DOT = dict(preferred_element_type=jnp.float32, precision=lax.Precision.HIGHEST)


def _mm_body(x_ref, w_ref, o_ref):
    o_ref[...] = lax.dot_general(x_ref[...], w_ref[...],
                                 (((1,), (0,)), ((), ())), **_DOT)


def _scale_body(deg_ref, m_ref, o_ref):
    dinv = lax.rsqrt(1.0 + deg_ref[0] + deg_ref[1])
    o_ref[...] = m_ref[...] * dinv


def _fused_body(agg_ref, h_ref, deg_ref, b_ref, w_ref, o_ref):
    dinv = lax.rsqrt(1.0 + deg_ref[0] + deg_ref[1])
    t = jnp.maximum(dinv * (agg_ref[0] + agg_ref[1] + h_ref[...]) + b_ref[...],
                    0.0)
    o_ref[...] = lax.dot_general(t * dinv, w_ref[...],
                                 (((1,), (0,)), ((), ())), **_DOT)


def _final_body(agg_ref, h_ref, deg_ref, b_ref, o_ref):
    dinv = lax.rsqrt(1.0 + deg_ref[0] + deg_ref[1])
    o_ref[...] = dinv * (agg_ref[0] + agg_ref[1] + h_ref[...]) + b_ref[...]


def _row_spec(d):
    return pl.BlockSpec((GB, d), lambda i: (i, 0))


_DEG_SPEC = pl.BlockSpec((NC, GB, 1), lambda i: (0, i, 0))


def _agg_spec(d):
    return pl.BlockSpec((NC, GB, d), lambda i: (0, i, 0))


def _w_spec(din, dout):
    return pl.BlockSpec((din, dout), lambda i: (0, 0))


def _b_spec(d):
    return pl.BlockSpec((1, d), lambda i: (0, 0))


_mm = pl.pallas_call(
    _mm_body, grid=(GRID,),
    in_specs=[_row_spec(128), _w_spec(128, 128)],
    out_specs=_row_spec(128),
    out_shape=jax.ShapeDtypeStruct((N, 128), jnp.float32),
)

_scale = pl.pallas_call(
    _scale_body, grid=(GRID,),
    in_specs=[_DEG_SPEC, _row_spec(128)],
    out_specs=_row_spec(128),
    out_shape=jax.ShapeDtypeStruct((N, 128), jnp.float32),
)


def _make_fused(dout):
    return pl.pallas_call(
        _fused_body, grid=(GRID,),
        in_specs=[_agg_spec(128), _row_spec(128), _DEG_SPEC, _b_spec(128),
                  _w_spec(128, dout)],
        out_specs=_row_spec(dout),
        out_shape=jax.ShapeDtypeStruct((N, dout), jnp.float32),
    )


_fused128 = _make_fused(128)
_fused64 = _make_fused(64)

_final = pl.pallas_call(
    _final_body, grid=(GRID,),
    in_specs=[_agg_spec(64), _row_spec(64), _DEG_SPEC, _b_spec(64)],
    out_specs=_row_spec(64),
    out_shape=jax.ShapeDtypeStruct((N, 64), jnp.float32),
)


def kernel(x, edge_index, W1, b1, W2, b2, W3, b3):
    pad = EP - E
    ar = jnp.arange(pad, dtype=jnp.int32)
    pad_src = (ar * 131) % N
    pad_dst = N + ar % (NA - N)
    src3 = jnp.concatenate([edge_index[0].astype(jnp.int32), pad_src])
    dst3 = jnp.concatenate([edge_index[1].astype(jnp.int32), pad_dst])
    src3 = src3.reshape(NW, STEPS, B)
    dst3 = dst3.reshape(NW, STEPS, B)
    deg2 = _deg(dst3)
    deg3 = deg2.reshape(NC, N, 1)
    mat1 = _mm(x, W1)
    h1p = _scale(deg3, mat1)
    agg1 = _agg128(h1p, src3, dst3)
    h2p = _fused128(agg1, h1p, deg3, b1.reshape(1, -1), W2)
    agg2 = _agg128(h2p, src3, dst3)
    h3p = _fused64(agg2, h2p, deg3, b2.reshape(1, -1), W3)
    agg3 = _agg64(h3p, src3, dst3)
    return _final(agg3, h3p, deg3, b3.reshape(1, -1))


# speedup vs baseline: 23.7517x; 1.0011x over previous
import functools

import jax
import jax.numpy as jnp
from jax import lax
from jax.experimental import pallas as pl
from jax.experimental.pallas import tpu as pltpu
from jax.experimental.pallas import tpu_sc as plsc

N = 10000
E = 320000
NC = 2
NS = 16
NW = NC * NS
B = 96
STEPS = 106
PAIRS = STEPS // 2
EP = NW * STEPS * B
NA = 10112
RPT = NA // NS
_CHUNKS = [B] * (RPT // B) + ([RPT % B] if RPT % B else [])

_MESH = plsc.VectorSubcoreMesh(core_axis_name="c", subcore_axis_name="s")
_SC_PARAMS = pltpu.CompilerParams(use_tc_tiling_on_sc=False)


@functools.partial(
    pl.kernel,
    out_type=jax.ShapeDtypeStruct((NC, N), jnp.float32),
    mesh=_MESH,
    scratch_types=[
        pltpu.VMEM((B,), jnp.float32),
        pltpu.VMEM((STEPS, B), jnp.int32),
        pltpu.VMEM((640,), jnp.float32),
        pltpu.VMEM_SHARED((NA,), jnp.float32),
    ],
    compiler_params=_SC_PARAMS,
)
def _deg(dst_hbm, deg_hbm, ones_v, idx_v, zbuf, acc):
    c = lax.axis_index("c")
    s = lax.axis_index("s")
    wid = c * NS + s
    one = jnp.ones((16,), jnp.float32)
    zero = jnp.zeros((16,), jnp.float32)
    for k in range(B // 16):
        ones_v[pl.ds(16 * k, 16)] = one

    @pl.when(s == 0)
    def _zero_acc():
        for k in range(40):
            zbuf[pl.ds(16 * k, 16)] = zero
        for k in range(NA // 640):
            pltpu.sync_copy(zbuf, acc.at[pl.ds(640 * k, 640)])
        rem = NA % 640
        if rem:
            pltpu.sync_copy(zbuf.at[pl.ds(0, rem)],
                            acc.at[pl.ds(NA - rem, rem)])

    pltpu.sync_copy(dst_hbm.at[wid], idx_v)
    plsc.subcore_barrier()

    def step(i, carry):
        pltpu.sync_copy(ones_v, acc.at[idx_v.at[i]], add=True)
        return carry

    lax.fori_loop(0, STEPS, step, 0)
    plsc.subcore_barrier()

    @pl.when(s == 0)
    def _copy_out():
        pltpu.sync_copy(acc.at[pl.ds(0, N)], deg_hbm.at[c])


def _make_agg(D):
    @functools.partial(
        pl.kernel,
        out_type=jax.ShapeDtypeStruct((NC, NA, D), jnp.float32),
        mesh=_MESH,
        scratch_types=[
            pltpu.VMEM((STEPS, B), jnp.int32),
            pltpu.VMEM((STEPS, B), jnp.int32),
            pltpu.VMEM((B, D), jnp.float32),
            pltpu.VMEM((B, D), jnp.float32),
            pltpu.VMEM_SHARED((NA, D), jnp.float32),
            pltpu.SemaphoreType.DMA,
            pltpu.SemaphoreType.DMA,
            pltpu.SemaphoreType.DMA,
            pltpu.SemaphoreType.DMA,
        ],
        compiler_params=_SC_PARAMS,
    )
    def agg(h_hbm, src_hbm, dst_hbm, out_hbm, sidx, didx, rows0, rows1, acc,
            gsem0, gsem1, ssem0, ssem1):
        c = lax.axis_index("c")
        s = lax.axis_index("s")
        wid = c * NS + s
        zero = jnp.zeros((16,), jnp.float32)

        def zrow(r, carry):
            def zcol(k, carry2):
                rows0[r, pl.ds(k * 16, 16)] = zero
                return carry2
            return lax.fori_loop(0, D // 16, zcol, carry)

        lax.fori_loop(0, B, zrow, 0)
        base = s * RPT
        off = 0
        for n in _CHUNKS:
            pltpu.sync_copy(rows0.at[pl.ds(0, n)],
                            acc.at[pl.ds(base + off, n)])
            off += n
        pltpu.sync_copy(src_hbm.at[wid], sidx)
        pltpu.sync_copy(dst_hbm.at[wid], didx)
        plsc.subcore_barrier()

        pltpu.async_copy(h_hbm.at[sidx.at[0]], rows0, gsem0)
        pltpu.async_copy(h_hbm.at[sidx.at[1]], rows1, gsem1)

        def pair(p, carry):
            i0 = 2 * p
            pltpu.make_async_copy(h_hbm.at[sidx.at[i0]], rows0, gsem0).wait()
            pltpu.async_copy(rows0, acc.at[didx.at[i0]], ssem0, add=True)
            pltpu.make_async_copy(h_hbm.at[sidx.at[i0 + 1]], rows1,
                                  gsem1).wait()
            pltpu.async_copy(rows1, acc.at[didx.at[i0 + 1]], ssem1, add=True)

            @pl.when(p < PAIRS - 1)
            def _prefetch():
                pltpu.make_async_copy(rows0, acc.at[didx.at[i0]],
                                      ssem0).wait()
                pltpu.async_copy(h_hbm.at[sidx.at[i0 + 2]], rows0, gsem0)
                pltpu.make_async_copy(rows1, acc.at[didx.at[i0 + 1]],
                                      ssem1).wait()
                pltpu.async_copy(h_hbm.at[sidx.at[i0 + 3]], rows1, gsem1)

            return carry

        lax.fori_loop(0, PAIRS, pair, 0)
        pltpu.make_async_copy(rows0, acc.at[didx.at[STEPS - 2]], ssem0).wait()
        pltpu.make_async_copy(rows1, acc.at[didx.at[STEPS - 1]], ssem1).wait()
        plsc.subcore_barrier()
        off = 0
        for n in _CHUNKS:
            pltpu.sync_copy(acc.at[pl.ds(base + off, n)],
                            out_hbm.at[c, pl.ds(base + off, n)])
            off += n

    return agg


_agg128 = _make_agg(128)
_agg64 = _make_agg(64)


GB = 2000
GRID = N // GB
_DOT = dict(preferred_element_type=jnp.float32, precision=lax.Precision.HIGHEST)


def _mm_body(x_ref, w_ref, o_ref):
    o_ref[...] = lax.dot_general(x_ref[...], w_ref[...],
                                 (((1,), (0,)), ((), ())), **_DOT)


def _scale_body(deg_ref, m_ref, o_ref):
    dinv = lax.rsqrt(1.0 + deg_ref[0] + deg_ref[1])
    o_ref[...] = m_ref[...] * dinv


def _fused_body(agg_ref, h_ref, deg_ref, b_ref, w_ref, o_ref):
    dinv = lax.rsqrt(1.0 + deg_ref[0] + deg_ref[1])
    t = jnp.maximum(dinv * (agg_ref[0] + agg_ref[1] + h_ref[...]) + b_ref[...],
                    0.0)
    o_ref[...] = lax.dot_general(t * dinv, w_ref[...],
                                 (((1,), (0,)), ((), ())), **_DOT)


def _final_body(agg_ref, h_ref, deg_ref, b_ref, o_ref):
    dinv = lax.rsqrt(1.0 + deg_ref[0] + deg_ref[1])
    o_ref[...] = dinv * (agg_ref[0] + agg_ref[1] + h_ref[...]) + b_ref[...]


def _row_spec(d):
    return pl.BlockSpec((GB, d), lambda i: (i, 0))


_DEG_SPEC = pl.BlockSpec((NC, GB, 1), lambda i: (0, i, 0))


def _agg_spec(d):
    return pl.BlockSpec((NC, GB, d), lambda i: (0, i, 0))


def _w_spec(din, dout):
    return pl.BlockSpec((din, dout), lambda i: (0, 0))


def _b_spec(d):
    return pl.BlockSpec((1, d), lambda i: (0, 0))


_mm = pl.pallas_call(
    _mm_body, grid=(GRID,),
    in_specs=[_row_spec(128), _w_spec(128, 128)],
    out_specs=_row_spec(128),
    out_shape=jax.ShapeDtypeStruct((N, 128), jnp.float32),
)

_scale = pl.pallas_call(
    _scale_body, grid=(GRID,),
    in_specs=[_DEG_SPEC, _row_spec(128)],
    out_specs=_row_spec(128),
    out_shape=jax.ShapeDtypeStruct((N, 128), jnp.float32),
)


def _make_fused(dout):
    return pl.pallas_call(
        _fused_body, grid=(GRID,),
        in_specs=[_agg_spec(128), _row_spec(128), _DEG_SPEC, _b_spec(128),
                  _w_spec(128, dout)],
        out_specs=_row_spec(dout),
        out_shape=jax.ShapeDtypeStruct((N, dout), jnp.float32),
    )


_fused128 = _make_fused(128)
_fused64 = _make_fused(64)

_final = pl.pallas_call(
    _final_body, grid=(GRID,),
    in_specs=[_agg_spec(64), _row_spec(64), _DEG_SPEC, _b_spec(64)],
    out_specs=_row_spec(64),
    out_shape=jax.ShapeDtypeStruct((N, 64), jnp.float32),
)


def kernel(x, edge_index, W1, b1, W2, b2, W3, b3):
    pad = EP - E
    ar = jnp.arange(pad, dtype=jnp.int32)
    pad_src = (ar * 131) % N
    pad_dst = N + ar % (NA - N)
    src3 = jnp.concatenate([edge_index[0].astype(jnp.int32), pad_src])
    dst3 = jnp.concatenate([edge_index[1].astype(jnp.int32), pad_dst])
    src3 = src3.reshape(NW, STEPS, B)
    dst3 = dst3.reshape(NW, STEPS, B)
    deg2 = _deg(dst3)
    deg3 = deg2.reshape(NC, N, 1)
    mat1 = _mm(x, W1)
    h1p = _scale(deg3, mat1)
    agg1 = _agg128(h1p, src3, dst3)
    h2p = _fused128(agg1, h1p, deg3, b1.reshape(1, -1), W2)
    agg2 = _agg128(h2p, src3, dst3)
    h3p = _fused64(agg2, h2p, deg3, b2.reshape(1, -1), W3)
    agg3 = _agg64(h3p, src3, dst3)
    return _final(agg3, h3p, deg3, b3.reshape(1, -1))
